# Initial kernel scaffold; baseline (speedup 1.0000x reference)
import jax
import jax.numpy as jnp
from jax.experimental import pallas as pl


def kernel(X, emb0, emb1, emb2, emb3, emb4, gamma0, beta0, mmean0, mvar0, W1, bb1, gamma1, beta1, mmean1, mvar1, W2, bb2, gamma2, beta2, mmean2, mvar2):
    raise NotImplementedError("write your pallas kernel here")


# speedup vs baseline: 2.1549x; 2.1549x over previous
import functools

import jax
import jax.numpy as jnp
from jax import lax
from jax.experimental import pallas as pl
from jax.experimental.pallas import tpu as pltpu
from jax.experimental.pallas import tpu_sc as plsc

N = 4096 * 20
NC, NS, LANES = 2, 16, 16
NW = NC * NS
RPW = N // NW
CHUNK = 512
IDX_ROW = 128
G = CHUNK // IDX_ROW
NCHUNK = RPW // CHUNK
D = 64
VOC_BIG = 100001
VOC_4 = 1001


def _sc_gather_body(x_hbm, e0, e1, e4, o0, o1, o4,
                    xv, i0, i1, i4, r0, r1, r4, sem):
    wid = lax.axis_index("s") * NC + lax.axis_index("c")
    lane = lax.iota(jnp.int32, LANES)

    def chunk_body(ci, carry):
        base = wid * RPW + ci * CHUNK
        pltpu.sync_copy(x_hbm.at[pl.ds(base * 5, CHUNK * 5)], xv)
        for g in range(G):
            for j in range(IDX_ROW // LANES):
                src = (g * IDX_ROW + j * LANES + lane) * 5
                sl = pl.ds(j * LANES, LANES)
                i0[g, sl] = lax.rem(plsc.load_gather(xv, [src]), VOC_BIG)
                i1[g, sl] = lax.rem(plsc.load_gather(xv, [src + 1]), VOC_BIG)
                i4[g, sl] = lax.rem(plsc.load_gather(xv, [src + 4]), VOC_4)
        cps = []
        for g in range(G):
            dst = pl.ds(g * IDX_ROW, IDX_ROW)
            cps.append(pltpu.async_copy(e0.at[i0.at[g]], r0.at[dst], sem))
            cps.append(pltpu.async_copy(e1.at[i1.at[g]], r1.at[dst], sem))
            cps.append(pltpu.async_copy(e4.at[i4.at[g]], r4.at[dst], sem))
        for c in cps:
            c.wait()
        rows = pl.ds(base, CHUNK)
        pltpu.sync_copy(r0, o0.at[rows])
        pltpu.sync_copy(r1, o1.at[rows])
        pltpu.sync_copy(r4, o4.at[rows])
        return carry

    lax.fori_loop(0, NCHUNK, chunk_body, 0)


def _sc_gather(xflat, emb0, emb1, emb4):
    mesh = plsc.VectorSubcoreMesh(core_axis_name="c", subcore_axis_name="s")
    out = jax.ShapeDtypeStruct((N, D), jnp.float32)
    run = pl.kernel(
        _sc_gather_body,
        out_type=(out, out, out),
        mesh=mesh,
        compiler_params=pltpu.CompilerParams(
            needs_layout_passes=False, use_tc_tiling_on_sc=False),
        scratch_types=[
            pltpu.VMEM((CHUNK * 5,), jnp.int32),
            pltpu.VMEM((G, IDX_ROW), jnp.int32),
            pltpu.VMEM((G, IDX_ROW), jnp.int32),
            pltpu.VMEM((G, IDX_ROW), jnp.int32),
            pltpu.VMEM((CHUNK, D), jnp.float32),
            pltpu.VMEM((CHUNK, D), jnp.float32),
            pltpu.VMEM((CHUNK, D), jnp.float32),
            pltpu.SemaphoreType.DMA,
        ],
    )
    return run(xflat, emb0, emb1, emb4)


TB = 512


def _elu(x):
    return jnp.where(x > 0, x, jnp.exp(jnp.minimum(x, 0.0)) - 1.0)


def _tc_dense_body(x_ref, g0_ref, g1_ref, g4_ref,
                   w0_ref, w1_ref, w4_ref, m23_ref,
                   w2_ref, b2_ref, s2_ref, t2_ref, o_ref):
    f32 = jnp.float32
    acc = jnp.dot(g0_ref[...], w0_ref[...], preferred_element_type=f32)
    acc += jnp.dot(g1_ref[...], w1_ref[...], preferred_element_type=f32)
    acc += jnp.dot(g4_ref[...], w4_ref[...], preferred_element_type=f32)
    x = x_ref[...]
    code = lax.rem(x[:, 2:3], 5) * 8 + lax.rem(x[:, 3:4], 8)
    oh = (code == lax.broadcasted_iota(jnp.int32, (TB, 40), 1)).astype(f32)
    acc += jnp.dot(oh, m23_ref[...], preferred_element_type=f32)
    a1 = _elu(acc)
    z2 = jnp.dot(a1, w2_ref[...], preferred_element_type=f32) + b2_ref[...]
    o_ref[...] = _elu(z2) * s2_ref[...] + t2_ref[...]


def _tc_dense(x2d, g0, g1, g4, w0, w1, w4, m23, w2, b2, s2, t2):
    row_spec = lambda c: pl.BlockSpec((TB, c), lambda i: (i, 0))
    full = lambda a: pl.BlockSpec(a.shape, lambda i: (0, 0))
    return pl.pallas_call(
        _tc_dense_body,
        grid=(N // TB,),
        in_specs=[
            row_spec(5), row_spec(D), row_spec(D), row_spec(D),
            full(w0), full(w1), full(w4), full(m23),
            full(w2), full(b2), full(s2), full(t2),
        ],
        out_specs=row_spec(100),
        out_shape=jax.ShapeDtypeStruct((N, 100), jnp.float32),
        compiler_params=pltpu.CompilerParams(
            dimension_semantics=("arbitrary",)),
    )(x2d, g0, g1, g4, w0, w1, w4, m23, w2, b2, s2, t2)


def kernel(X, emb0, emb1, emb2, emb3, emb4,
           gamma0, beta0, mmean0, mvar0,
           W1, bb1,
           gamma1, beta1, mmean1, mvar1,
           W2, bb2,
           gamma2, beta2, mmean2, mvar2):
    s0 = gamma0 * lax.rsqrt(mvar0 + 1e-3)
    t0 = beta0 - mmean0 * s0
    W1p = W1 * s0[:, None]
    b1p = t0 @ W1 + bb1
    s1 = gamma1 * lax.rsqrt(mvar1 + 1e-3)
    t1 = beta1 - mmean1 * s1
    W2p = W2 * s1[:, None]
    b2p = t1 @ W2 + bb2
    s2 = gamma2 * lax.rsqrt(mvar2 + 1e-3)
    t2 = beta2 - mmean2 * s2
    m23 = ((emb2 @ W1p[100:103])[:, None, :]
           + (emb3 @ W1p[103:107])[None, :, :]).reshape(40, 150)
    m23 = m23 + b1p[None, :]

    xflat = X.reshape(-1)
    x2d = X.reshape(N, 5)
    padt = lambda e: jnp.pad(e, ((0, 0), (0, D - 50)))
    padw = lambda w: jnp.pad(w, ((0, D - 50), (0, 0)))
    g0, g1, g4 = _sc_gather(xflat, padt(emb0), padt(emb1), padt(emb4))
    out = _tc_dense(
        x2d, g0, g1, g4,
        padw(W1p[0:50]), padw(W1p[50:100]), padw(W1p[107:157]), m23,
        W2p, b2p.reshape(1, 100), s2.reshape(1, 100), t2.reshape(1, 100))
    return out.reshape(4096, 20, 100)
